# baseline (speedup 1.0000x reference)
import functools

import jax
import jax.numpy as jnp
from jax import lax
from jax.experimental import pallas as pl
from jax.experimental.pallas import tpu as pltpu
from jax.experimental.pallas import tpu_sc as plsc

LANES = 16
CHUNK = 128
EPS = 1e-12

_info = plsc.get_sparse_core_info()
_NC, _NS = _info.num_cores, _info.num_subcores
NW = _NC * _NS


def _rsqrt(x):
    i = lax.bitcast_convert_type(x, jnp.int32)
    i = jnp.int32(0x5F3759DF) - (i >> 1)
    y = lax.bitcast_convert_type(i, jnp.float32)
    for _ in range(3):
        y = y * (1.5 - 0.5 * x * y * y)
    return y


def _make_sc_kernel(N, S, H):
    NH = H // LANES
    per_w = N // NW
    nchunks = per_w // CHUNK
    assert nchunks % 2 == 0 and per_w % CHUNK == 0
    inv_h = 1.0 / H

    def body(ids_hbm, tt_hbm, word_hbm, pos_hbm, consts_hbm, out_hbm,
             ids_v, tt_v, rows_v, out_v, pos_v, cv, sg0, sg1, so0, so1):
        wid = lax.axis_index("s") * _NC + lax.axis_index("c")
        w0 = wid * per_w
        pltpu.sync_copy(pos_hbm, pos_v)
        pltpu.sync_copy(consts_hbm, cv)
        sem_g = (sg0, sg1)
        sem_o = (so0, so1)

        iota = lax.iota(jnp.int32, LANES)
        lane_idx = [jnp.full((LANES, 1), k, jnp.int32) for k in range(LANES)]
        xor_idx = {d: lax.reshape(iota ^ d, (LANES, 1)) for d in (8, 4, 2, 1)}
        gd = lax.GatherDimensionNumbers(
            offset_dims=(), collapsed_slice_dims=(0,), start_index_map=(0,))

        def perm(vec, idx):
            return lax.gather(vec, idx, gd, slice_sizes=(1,),
                              mode=lax.GatherScatterMode.PROMISE_IN_BOUNDS)

        def lane_sum(vec):
            for d in (8, 4, 2, 1):
                vec = vec + perm(vec, xor_idx[d])
            return vec

        def prep(c, b):
            pltpu.sync_copy(ids_hbm.at[pl.ds(w0 + c * CHUNK, CHUNK)],
                            ids_v.at[b])
            pltpu.sync_copy(tt_hbm.at[pl.ds(w0 + c * CHUNK, CHUNK)],
                            tt_v.at[b])
            pltpu.async_copy(word_hbm.at[ids_v.at[b]], rows_v.at[b], sem_g[b])

        prep(0, 0)

        def pair_body(i, carry):
            for b in range(2):
                c = i * 2 + b
                bn = 1 - b

                @pl.when(c + 1 < nchunks)
                def _():
                    prep(c + 1, bn)

                pltpu.make_async_copy(word_hbm.at[ids_v.at[b]], rows_v.at[b],
                                      sem_g[b]).wait()

                @pl.when(c >= 2)
                def _():
                    pltpu.make_async_copy(out_v.at[b],
                                          out_hbm.at[pl.ds(0, CHUNK)],
                                          sem_o[b]).wait()

                p0 = (w0 + c * CHUNK) % S

                def grp_body(g, tc):
                    tb = g * LANES
                    tg = tt_v[b, pl.ds(tb, LANES)]
                    for k in range(LANES):
                        t = tb + k
                        p = p0 + t
                        p = jnp.where(p >= S, p - S, p)
                        ttf = perm(tg, lane_idx[k]).astype(jnp.float32)
                        xs = []
                        s = None
                        q = None
                        for j in range(NH):
                            sl = pl.ds(j * LANES, LANES)
                            x = rows_v[b, t, sl] + pos_v[p, sl] + ttf * cv[0, sl]
                            xs.append(x)
                            s = x if s is None else s + x
                            q = x * x if q is None else q + x * x
                        m = lane_sum(s) * inv_h
                        var = jnp.maximum(lane_sum(q) * inv_h - m * m, 0.0)
                        r = _rsqrt(var + EPS)
                        for j in range(NH):
                            sl = pl.ds(j * LANES, LANES)
                            out_v[b, t, sl] = ((xs[j] - m) * r * cv[1, sl]
                                               + cv[2, sl])
                    return tc

                lax.fori_loop(0, CHUNK // LANES, grp_body, 0)
                pltpu.async_copy(out_v.at[b],
                                 out_hbm.at[pl.ds(w0 + c * CHUNK, CHUNK)],
                                 sem_o[b])
            return carry

        lax.fori_loop(0, nchunks // 2, pair_body, 0)
        pltpu.make_async_copy(out_v.at[0], out_hbm.at[pl.ds(0, CHUNK)],
                              sem_o[0]).wait()
        pltpu.make_async_copy(out_v.at[1], out_hbm.at[pl.ds(0, CHUNK)],
                              sem_o[1]).wait()

    return pl.kernel(
        body,
        out_type=jax.ShapeDtypeStruct((N, H), jnp.float32),
        mesh=plsc.VectorSubcoreMesh(core_axis_name="c", subcore_axis_name="s"),
        scratch_types=[
            pltpu.VMEM((2, CHUNK), jnp.int32),
            pltpu.VMEM((2, CHUNK), jnp.int32),
            pltpu.VMEM((2, CHUNK, H), jnp.float32),
            pltpu.VMEM((2, CHUNK, H), jnp.float32),
            pltpu.VMEM((S, H), jnp.float32),
            pltpu.VMEM((3, H), jnp.float32),
            pltpu.SemaphoreType.DMA,
            pltpu.SemaphoreType.DMA,
            pltpu.SemaphoreType.DMA,
            pltpu.SemaphoreType.DMA,
        ],
    )


def kernel(input_ids, token_type_ids, word_table, pos_table, type_table,
           ln_gamma, ln_beta):
    B, S = input_ids.shape
    H = word_table.shape[1]
    N = B * S
    ids = input_ids.reshape(N).astype(jnp.int32)
    tt = token_type_ids.reshape(N).astype(jnp.int32)
    pos = pos_table[:S] + type_table[0]
    consts = jnp.stack([type_table[1] - type_table[0], ln_gamma, ln_beta])
    out = _make_sc_kernel(N, S, H)(ids, tt, word_table, pos, consts)
    return out.reshape(B, S, H)


# speedup vs baseline: 5.3586x; 5.3586x over previous
import functools

import jax
import jax.numpy as jnp
from jax import lax
from jax.experimental import pallas as pl
from jax.experimental.pallas import tpu as pltpu
from jax.experimental.pallas import tpu_sc as plsc

CHUNK = 128
BB = 16
EPS = 1e-12

_info = plsc.get_sparse_core_info()
_NC, _NS = _info.num_cores, _info.num_subcores
NW = _NC * _NS


def _make_sc_gather(N, H):
    per_w = N // NW
    nchunks = per_w // CHUNK
    assert nchunks % 2 == 0 and per_w % CHUNK == 0

    def body(ids_hbm, word_hbm, out_hbm, ids_v, rows_v, sg0, sg1, so0, so1):
        wid = lax.axis_index("s") * _NC + lax.axis_index("c")
        w0 = wid * per_w
        sem_g = (sg0, sg1)
        sem_o = (so0, so1)

        def prep(c, b):
            pltpu.sync_copy(ids_hbm.at[pl.ds(w0 + c * CHUNK, CHUNK)],
                            ids_v.at[b])
            pltpu.async_copy(word_hbm.at[ids_v.at[b]], rows_v.at[b], sem_g[b])

        prep(0, 0)

        def pair_body(i, carry):
            for b in range(2):
                c = i * 2 + b
                bn = 1 - b
                pltpu.make_async_copy(word_hbm.at[ids_v.at[b]], rows_v.at[b],
                                      sem_g[b]).wait()
                pltpu.async_copy(rows_v.at[b],
                                 out_hbm.at[pl.ds(w0 + c * CHUNK, CHUNK)],
                                 sem_o[b])

                @pl.when(c + 1 < nchunks)
                def _():
                    prep(c + 1, bn)
                    @pl.when(c >= 1)
                    def _():
                        pltpu.make_async_copy(rows_v.at[bn],
                                              out_hbm.at[pl.ds(0, CHUNK)],
                                              sem_o[bn]).wait()
                    pltpu.async_copy(word_hbm.at[ids_v.at[bn]],
                                     rows_v.at[bn], sem_g[bn])
            return carry

        lax.fori_loop(0, nchunks // 2, pair_body, 0)
        pltpu.make_async_copy(rows_v.at[0], out_hbm.at[pl.ds(0, CHUNK)],
                              sem_o[0]).wait()
        pltpu.make_async_copy(rows_v.at[1], out_hbm.at[pl.ds(0, CHUNK)],
                              sem_o[1]).wait()

    def prep_fixed(c, b):
        pass

    return pl.kernel(
        body,
        out_type=jax.ShapeDtypeStruct((N, H), jnp.float32),
        mesh=plsc.VectorSubcoreMesh(core_axis_name="c", subcore_axis_name="s"),
        scratch_types=[
            pltpu.VMEM((2, CHUNK), jnp.int32),
            pltpu.VMEM((2, CHUNK, H), jnp.float32),
            pltpu.SemaphoreType.DMA,
            pltpu.SemaphoreType.DMA,
            pltpu.SemaphoreType.DMA,
            pltpu.SemaphoreType.DMA,
        ],
    )


def _tc_body(x_ref, tt_ref, pos_ref, ty_ref, gb_ref, o_ref):
    x = x_ref[...]
    ttf = tt_ref[...].astype(jnp.float32)
    pos = pos_ref[...]
    tdiff = ty_ref[1] - ty_ref[0]
    gamma = gb_ref[0]
    beta = gb_ref[1]
    x = x + (pos[None, :, :] + ty_ref[0]) + ttf * tdiff
    mean = jnp.mean(x, axis=-1, keepdims=True)
    xc = x - mean
    var = jnp.mean(xc * xc, axis=-1, keepdims=True)
    o_ref[...] = xc * lax.rsqrt(var + EPS) * gamma + beta


def _tc_ln(B, S, H):
    grid = (B // BB,)
    return pl.pallas_call(
        _tc_body,
        grid=grid,
        in_specs=[
            pl.BlockSpec((BB, S, H), lambda i: (i, 0, 0)),
            pl.BlockSpec((BB, S, 1), lambda i: (i, 0, 0)),
            pl.BlockSpec((S, H), lambda i: (0, 0)),
            pl.BlockSpec((2, H), lambda i: (0, 0)),
            pl.BlockSpec((2, H), lambda i: (0, 0)),
        ],
        out_specs=pl.BlockSpec((BB, S, H), lambda i: (i, 0, 0)),
        out_shape=jax.ShapeDtypeStruct((B, S, H), jnp.float32),
    )


def kernel(input_ids, token_type_ids, word_table, pos_table, type_table,
           ln_gamma, ln_beta):
    B, S = input_ids.shape
    H = word_table.shape[1]
    N = B * S
    ids = input_ids.reshape(N).astype(jnp.int32)
    tt = token_type_ids.astype(jnp.int32)[:, :, None]
    gathered = _make_sc_gather(N, H)(ids, word_table).reshape(B, S, H)
    gb = jnp.stack([ln_gamma, ln_beta])
    return _tc_ln(B, S, H)(gathered, tt, pos_table[:S], type_table, gb)


# speedup vs baseline: 5.7515x; 1.0733x over previous
import functools

import jax
import jax.numpy as jnp
from jax import lax
from jax.experimental import pallas as pl
from jax.experimental.pallas import tpu as pltpu
from jax.experimental.pallas import tpu_sc as plsc

CHUNK = 128
BB = 16
EPS = 1e-12

_info = plsc.get_sparse_core_info()
_NC, _NS = _info.num_cores, _info.num_subcores
NW = _NC * _NS


def _make_sc_gather(N, H):
    per_w = N // NW
    nchunks = per_w // CHUNK
    assert nchunks % 2 == 0 and per_w % CHUNK == 0

    def body(ids_hbm, word_hbm, out_hbm, ids_v, rows_v, sg0, sg1, so0, so1):
        wid = lax.axis_index("s") * _NC + lax.axis_index("c")
        w0 = wid * per_w
        sem_g = (sg0, sg1)
        sem_o = (so0, so1)

        pltpu.sync_copy(ids_hbm.at[pl.ds(w0, CHUNK)], ids_v.at[0])
        pltpu.async_copy(word_hbm.at[ids_v.at[0]], rows_v.at[0], sem_g[0])

        def pair_body(i, carry):
            for b in range(2):
                c = i * 2 + b
                bn = 1 - b
                pltpu.make_async_copy(word_hbm.at[ids_v.at[b]], rows_v.at[b],
                                      sem_g[b]).wait()
                pltpu.async_copy(rows_v.at[b],
                                 out_hbm.at[pl.ds(w0 + c * CHUNK, CHUNK)],
                                 sem_o[b])

                @pl.when(c + 1 < nchunks)
                def _():
                    pltpu.sync_copy(
                        ids_hbm.at[pl.ds(w0 + (c + 1) * CHUNK, CHUNK)],
                        ids_v.at[bn])
                    @pl.when(c >= 1)
                    def _():
                        pltpu.make_async_copy(rows_v.at[bn],
                                              out_hbm.at[pl.ds(0, CHUNK)],
                                              sem_o[bn]).wait()
                    pltpu.async_copy(word_hbm.at[ids_v.at[bn]],
                                     rows_v.at[bn], sem_g[bn])
            return carry

        lax.fori_loop(0, nchunks // 2, pair_body, 0)
        pltpu.make_async_copy(rows_v.at[0], out_hbm.at[pl.ds(0, CHUNK)],
                              sem_o[0]).wait()
        pltpu.make_async_copy(rows_v.at[1], out_hbm.at[pl.ds(0, CHUNK)],
                              sem_o[1]).wait()

    def prep_fixed(c, b):
        pass

    return pl.kernel(
        body,
        out_type=jax.ShapeDtypeStruct((N, H), jnp.float32),
        mesh=plsc.VectorSubcoreMesh(core_axis_name="c", subcore_axis_name="s"),
        scratch_types=[
            pltpu.VMEM((2, CHUNK), jnp.int32),
            pltpu.VMEM((2, CHUNK, H), jnp.float32),
            pltpu.SemaphoreType.DMA,
            pltpu.SemaphoreType.DMA,
            pltpu.SemaphoreType.DMA,
            pltpu.SemaphoreType.DMA,
        ],
    )


def _tc_body(x_ref, tt_ref, pos_ref, ty_ref, gb_ref, o_ref):
    x = x_ref[...]
    ttf = tt_ref[...].astype(jnp.float32)
    pos = pos_ref[...]
    tdiff = ty_ref[1] - ty_ref[0]
    gamma = gb_ref[0]
    beta = gb_ref[1]
    x = x + (pos[None, :, :] + ty_ref[0]) + ttf * tdiff
    mean = jnp.mean(x, axis=-1, keepdims=True)
    xc = x - mean
    var = jnp.mean(xc * xc, axis=-1, keepdims=True)
    o_ref[...] = xc * lax.rsqrt(var + EPS) * gamma + beta


def _tc_ln(B, S, H):
    grid = (B // BB,)
    return pl.pallas_call(
        _tc_body,
        grid=grid,
        in_specs=[
            pl.BlockSpec((BB, S, H), lambda i: (i, 0, 0)),
            pl.BlockSpec((BB, S, 1), lambda i: (i, 0, 0)),
            pl.BlockSpec((S, H), lambda i: (0, 0)),
            pl.BlockSpec((2, H), lambda i: (0, 0)),
            pl.BlockSpec((2, H), lambda i: (0, 0)),
        ],
        out_specs=pl.BlockSpec((BB, S, H), lambda i: (i, 0, 0)),
        out_shape=jax.ShapeDtypeStruct((B, S, H), jnp.float32),
    )


def kernel(input_ids, token_type_ids, word_table, pos_table, type_table,
           ln_gamma, ln_beta):
    B, S = input_ids.shape
    H = word_table.shape[1]
    N = B * S
    ids = input_ids.reshape(N).astype(jnp.int32)
    tt = token_type_ids.astype(jnp.int32)[:, :, None]
    gathered = _make_sc_gather(N, H)(ids, word_table).reshape(B, S, H)
    gb = jnp.stack([ln_gamma, ln_beta])
    return _tc_ln(B, S, H)(gathered, tt, pos_table[:S], type_table, gb)


# speedup vs baseline: 6.7047x; 1.1657x over previous
import functools

import jax
import jax.numpy as jnp
from jax import lax
from jax.experimental import pallas as pl
from jax.experimental.pallas import tpu as pltpu
from jax.experimental.pallas import tpu_sc as plsc

CHUNK = 128
NBUF = 4
BB = 64
EPS = 1e-12

_info = plsc.get_sparse_core_info()
_NC, _NS = _info.num_cores, _info.num_subcores
NW = _NC * _NS


def _make_sc_gather(N, H):
    per_w = N // NW
    nchunks = per_w // CHUNK
    assert per_w % CHUNK == 0 and nchunks % 2 == 0 and nchunks >= NBUF

    def body(ids_hbm, word_hbm, out_hbm, ids_v, rows_v, *sems):
        sem_g = sems[:NBUF]
        sem_o = sems[NBUF:]
        wid = lax.axis_index("s") * _NC + lax.axis_index("c")
        w0 = wid * per_w
        pltpu.sync_copy(ids_hbm.at[wid], ids_v)

        def gather(c, b):
            pltpu.async_copy(word_hbm.at[ids_v.at[c]], rows_v.at[b], sem_g[b])

        def gather_wait(b):
            pltpu.make_async_copy(word_hbm.at[ids_v.at[0]], rows_v.at[b],
                                  sem_g[b]).wait()

        def put(c, b):
            pltpu.async_copy(rows_v.at[b],
                             out_hbm.at[pl.ds(w0 + c * CHUNK, CHUNK)],
                             sem_o[b])

        def put_wait(b):
            pltpu.make_async_copy(rows_v.at[b], out_hbm.at[pl.ds(0, CHUNK)],
                                  sem_o[b]).wait()

        gather(0, 0)
        gather(1, 1)

        nmain = (nchunks - 2) // NBUF * NBUF

        def quad_body(i, carry):
            for k in range(NBUF):
                c = i * NBUF + k
                gather_wait(k)
                put(c, k)
                b2 = (k + 2) % NBUF
                if k >= 2:
                    put_wait(b2)
                else:
                    @pl.when(i >= 1)
                    def _():
                        put_wait(b2)
                gather(c + 2, b2)
            return carry

        lax.fori_loop(0, nmain // NBUF, quad_body, 0)
        for c in range(nmain, nchunks):
            b = c % NBUF
            gather_wait(b)
            put(c, b)
            if c + 2 < nchunks:
                b2 = (c + 2) % NBUF
                put_wait(b2)
                gather(c + 2, b2)
        for c in range(nchunks - NBUF, nchunks):
            put_wait(c % NBUF)

    return pl.kernel(
        body,
        out_type=jax.ShapeDtypeStruct((N, H), jnp.float32),
        mesh=plsc.VectorSubcoreMesh(core_axis_name="c", subcore_axis_name="s"),
        scratch_types=[
            pltpu.VMEM((nchunks, CHUNK), jnp.int32),
            pltpu.VMEM((NBUF, CHUNK, H), jnp.float32),
        ] + [pltpu.SemaphoreType.DMA] * (2 * NBUF),
    )


def _tc_body(x_ref, tt_ref, pos_ref, ty_ref, gb_ref, o_ref):
    x = x_ref[...]
    ttf = tt_ref[...].astype(jnp.float32)
    pos = pos_ref[...]
    tdiff = ty_ref[1] - ty_ref[0]
    gamma = gb_ref[0]
    beta = gb_ref[1]
    x = x + (pos[None, :, :] + ty_ref[0]) + ttf * tdiff
    mean = jnp.mean(x, axis=-1, keepdims=True)
    xc = x - mean
    var = jnp.mean(xc * xc, axis=-1, keepdims=True)
    o_ref[...] = xc * lax.rsqrt(var + EPS) * gamma + beta


def _tc_ln(B, S, H):
    grid = (B // BB,)
    return pl.pallas_call(
        _tc_body,
        grid=grid,
        in_specs=[
            pl.BlockSpec((BB, S, H), lambda i: (i, 0, 0)),
            pl.BlockSpec((BB, S, 1), lambda i: (i, 0, 0)),
            pl.BlockSpec((S, H), lambda i: (0, 0)),
            pl.BlockSpec((2, H), lambda i: (0, 0)),
            pl.BlockSpec((2, H), lambda i: (0, 0)),
        ],
        out_specs=pl.BlockSpec((BB, S, H), lambda i: (i, 0, 0)),
        out_shape=jax.ShapeDtypeStruct((B, S, H), jnp.float32),
    )


def kernel(input_ids, token_type_ids, word_table, pos_table, type_table,
           ln_gamma, ln_beta):
    B, S = input_ids.shape
    H = word_table.shape[1]
    N = B * S
    ids = input_ids.reshape(NW, N // (NW * CHUNK), CHUNK).astype(jnp.int32)
    tt = token_type_ids.astype(jnp.int32)[:, :, None]
    gathered = _make_sc_gather(N, H)(ids, word_table).reshape(B, S, H)
    gb = jnp.stack([ln_gamma, ln_beta])
    return _tc_ln(B, S, H)(gathered, tt, pos_table[:S], type_table, gb)
